# baseline (speedup 1.0000x reference)
import functools

import jax
import jax.numpy as jnp
from jax import lax
from jax.experimental import pallas as pl
from jax.experimental.pallas import tpu as pltpu
from jax.experimental.pallas import tpu_sc as plsc


def _make_sc_kernel(B, S, H, V):
    NC, NS = 2, 16
    NW = NC * NS
    BW = B // NW
    G = H * 4
    VQ = V // 4
    assert BW == 128 and G == 128 and H == 32

    mesh = plsc.VectorSubcoreMesh(core_axis_name="c", subcore_axis_name="s")

    @functools.partial(
        pl.kernel,
        mesh=mesh,
        compiler_params=pltpu.CompilerParams(
            use_tc_tiling_on_sc=True, needs_layout_passes=False
        ),
        out_type=jax.ShapeDtypeStruct((S, H, B), jnp.float32),
        scratch_types=[
            pltpu.VMEM((S, BW), jnp.int32),
            pltpu.VMEM((BW,), jnp.int32),
            pltpu.VMEM((BW,), jnp.int32),
            pltpu.VMEM((BW, G), jnp.float32),
            pltpu.VMEM((BW, G), jnp.float32),
            pltpu.VMEM((H, BW), jnp.float32),
            pltpu.VMEM((H, BW), jnp.float32),
            pltpu.VMEM((S, G), jnp.float32),
            pltpu.SemaphoreType.DMA,
            pltpu.SemaphoreType.DMA,
            pltpu.SemaphoreType.DMA,
            pltpu.SemaphoreType.DMA,
        ],
    )
    def k(xt_hbm, embq_hbm, pos4_hbm, out_hbm, idx_v, q0, q1, gb0, gb1,
          ob0, ob1, pos_v, g0, g1, w0, w1):
        qs = (q0, q1)
        gbs = (gb0, gb1)
        obs = (ob0, ob1)
        gsems = (g0, g1)
        wsems = (w0, w1)
        wid = lax.axis_index("s") * NC + lax.axis_index("c")
        bw0 = pl.multiple_of(wid * BW, BW)
        pltpu.sync_copy(xt_hbm.at[:, pl.ds(bw0, BW)], idx_v)
        pltpu.sync_copy(pos4_hbm, pos_v)

        iota16 = lax.iota(jnp.int32, 16)

        def make_qidx(s, slot):
            for g in range(BW // 16):
                sl = pl.ds(g * 16, 16)
                qs[slot][sl] = lax.shift_right_logical(idx_v[s, sl], 2)

        def start_gather(s, slot):
            pltpu.async_copy(embq_hbm.at[qs[slot]], gbs[slot], gsems[slot])

        def extract_block(s, slot):
            gb = gbs[slot]
            ob = obs[slot]
            bases = []
            for g in range(BW // 16):
                tv = idx_v[s, pl.ds(g * 16, 16)]
                toff = lax.shift_left(jnp.bitwise_and(tv, 3), 5)
                bases.append(toff)

            srows = jnp.full((16,), s, jnp.int32)

            def h_body(h, carry):
                pv = plsc.load_gather(
                    pos_v, [srows, jnp.full((16,), h, jnp.int32)]
                )
                for g in range(BW // 16):
                    rows = iota16 + (g * 16)
                    cols = bases[g] + h
                    vals = plsc.load_gather(gb, [rows, cols])
                    ob[h, pl.ds(g * 16, 16)] = vals + pv
                return carry

            lax.fori_loop(0, H, h_body, 0)

        def start_write(s, slot):
            pltpu.async_copy(
                obs[slot], out_hbm.at[s, :, pl.ds(bw0, BW)], wsems[slot]
            )

        def wait_write(slot):
            pltpu.make_async_copy(
                obs[slot], out_hbm.at[0, :, pl.ds(bw0, BW)], wsems[slot]
            ).wait()

        def wait_gather(slot):
            pltpu.make_async_copy(
                embq_hbm.at[pl.ds(0, BW)], gbs[slot], gsems[slot]
            ).wait()

        make_qidx(0, 0)
        start_gather(0, 0)

        def pair_body(p, carry):
            for i in range(2):
                s = p * 2 + i

                @pl.when(s + 1 < S)
                def _():
                    make_qidx(s + 1, 1 - i)
                    start_gather(s + 1, 1 - i)

                wait_gather(i)

                @pl.when(s >= 2)
                def _():
                    wait_write(i)

                extract_block(s, i)
                start_write(s, i)
            return carry

        lax.fori_loop(0, S // 2, pair_body, 0)
        wait_write(0)
        wait_write(1)

    return k


def kernel(x, emb, pos_emb):
    B, S = x.shape
    V, H = emb.shape
    k = _make_sc_kernel(B, S, H, V)
    xt = x.T.astype(jnp.int32)
    embq = emb.reshape(V // 4, H * 4)
    pos4 = jnp.tile(pos_emb, (1, 4))
    out = k(xt, embq, pos4)
    return out.transpose(2, 0, 1)
